# baseline (speedup 1.0000x reference)
import functools

import jax
import jax.numpy as jnp
from jax import lax
from jax.experimental import pallas as pl
from jax.experimental.pallas import tpu as pltpu
from jax.experimental.pallas import tpu_sc as plsc

N = 10000
E = 320000
NODE_DIM = 128
EDGE_DIM = 16
H = 16
STEPS = 3

NC = 2
NS = 16
NW = NC * NS
PER_TILE = E // NW
CHUNK = 1000

A_BLK = 8000
MSG_BLK = 8000

_f32 = jnp.float32
_bf16 = jnp.bfloat16

_SC_PARAMS = pltpu.CompilerParams(use_tc_tiling_on_sc=False)


def _gather_body(h_hbm, src_hbm, out_hbm, idx_v, rows_v, sem):
    wid = lax.axis_index("s") * NC + lax.axis_index("c")
    base = wid * PER_TILE

    @pl.loop(0, PER_TILE, step=CHUNK)
    def _(off):
        pltpu.sync_copy(src_hbm.at[pl.ds(base + off, CHUNK)], idx_v)
        pltpu.async_copy(h_hbm.at[idx_v], rows_v, sem).wait()
        pltpu.sync_copy(rows_v, out_hbm.at[pl.ds(base + off, CHUNK)])


def _sc_gather(h, src):
    mesh = plsc.VectorSubcoreMesh(core_axis_name="c", subcore_axis_name="s")
    k = pl.kernel(
        _gather_body,
        out_type=jax.ShapeDtypeStruct((E, H), _f32),
        mesh=mesh,
        scratch_types=[
            pltpu.VMEM((CHUNK,), jnp.int32),
            pltpu.VMEM((CHUNK, H), _f32),
            pltpu.SemaphoreType.DMA,
        ],
        compiler_params=_SC_PARAMS,
    )
    return k(h, src)


def _scatter_body(msg_hbm, dst_hbm, zero_hbm, out_hbm, idx_v, rows_v, acc_sh, sem):
    cid = lax.axis_index("c")
    sid = lax.axis_index("s")
    wid = sid * NC + cid

    @pl.when(sid == 0)
    def _():
        pltpu.sync_copy(zero_hbm, acc_sh)

    plsc.subcore_barrier()

    base = wid * PER_TILE

    @pl.loop(0, PER_TILE, step=CHUNK)
    def _(off):
        pltpu.sync_copy(dst_hbm.at[pl.ds(base + off, CHUNK)], idx_v)
        pltpu.sync_copy(msg_hbm.at[pl.ds(base + off, CHUNK)], rows_v)
        pltpu.sync_copy(rows_v, acc_sh.at[idx_v], add=True)

    plsc.subcore_barrier()

    rows_per = N // NS
    pltpu.sync_copy(
        acc_sh.at[pl.ds(sid * rows_per, rows_per)],
        out_hbm.at[cid, pl.ds(sid * rows_per, rows_per)],
    )


def _sc_scatter(msg, dst, zeros):
    mesh = plsc.VectorSubcoreMesh(core_axis_name="c", subcore_axis_name="s")
    k = pl.kernel(
        _scatter_body,
        out_type=jax.ShapeDtypeStruct((NC, N, H), _f32),
        mesh=mesh,
        scratch_types=[
            pltpu.VMEM((CHUNK,), jnp.int32),
            pltpu.VMEM((CHUNK, H), _f32),
            pltpu.VMEM_SHARED((N, H), _f32),
            pltpu.SemaphoreType.DMA,
        ],
        compiler_params=_SC_PARAMS,
    )
    return k(msg, dst, zeros)


def _proj_body(x_ref, w_ref, b_ref, o_ref, or_ref):
    h = jnp.dot(x_ref[...], w_ref[...], preferred_element_type=_f32) + b_ref[...]
    o_ref[...] = h
    or_ref[...] = h.astype(_bf16).astype(_f32)


def _tc_proj(node_feat, W_proj_t, b_proj_row):
    return pl.pallas_call(
        _proj_body,
        out_shape=(
            jax.ShapeDtypeStruct((N, H), _f32),
            jax.ShapeDtypeStruct((N, H), _f32),
        ),
    )(node_feat, W_proj_t, b_proj_row)


def _edgeA_body(ef_ref, w_ref, b_ref, o_ref):
    a = jnp.dot(ef_ref[...], w_ref[...], preferred_element_type=_f32) + b_ref[...]
    o_ref[...] = a.astype(_bf16)


def _tc_edgeA(ef, We_t, b_edge_row):
    grid = (E // A_BLK,)
    return pl.pallas_call(
        _edgeA_body,
        grid=grid,
        compiler_params=pltpu.CompilerParams(
            dimension_semantics=("parallel",)),
        in_specs=[
            pl.BlockSpec((A_BLK, EDGE_DIM), lambda i: (i, 0)),
            pl.BlockSpec((EDGE_DIM, H * H), lambda i: (0, 0)),
            pl.BlockSpec((1, H * H), lambda i: (0, 0)),
        ],
        out_specs=pl.BlockSpec((A_BLK, H * H), lambda i: (i, 0)),
        out_shape=jax.ShapeDtypeStruct((E, H * H), _bf16),
    )(ef, We_t, b_edge_row)


def _msg_body(hs_ref, a_ref, rt_ref, s_ref, o_ref):
    hs = hs_ref[...].astype(_bf16)
    a = a_ref[...]
    rt = rt_ref[...]
    s = s_ref[...]
    t = jnp.dot(hs, rt, preferred_element_type=_f32)
    p = a.astype(_f32) * t
    p_hi = p.astype(_bf16)
    p_lo = (p - p_hi.astype(_f32)).astype(_bf16)
    o_ref[...] = (
        jnp.dot(p_hi, s, preferred_element_type=_f32)
        + jnp.dot(p_lo, s, preferred_element_type=_f32)
    )


def _tc_msg(h_src, A_b, RT, S):
    grid = (E // MSG_BLK,)
    return pl.pallas_call(
        _msg_body,
        grid=grid,
        compiler_params=pltpu.CompilerParams(
            dimension_semantics=("parallel",)),
        in_specs=[
            pl.BlockSpec((MSG_BLK, H), lambda i: (i, 0)),
            pl.BlockSpec((MSG_BLK, H * H), lambda i: (i, 0)),
            pl.BlockSpec((H, H * H), lambda i: (0, 0)),
            pl.BlockSpec((H * H, H), lambda i: (0, 0)),
        ],
        out_specs=pl.BlockSpec((MSG_BLK, H), lambda i: (i, 0)),
        out_shape=jax.ShapeDtypeStruct((E, H), _f32),
    )(h_src, A_b, RT, S)


def _gru_body(mp_ref, h_ref, wih_ref, whh_ref, bih_ref, bhh_ref, o_ref, or_ref):
    m = mp_ref[0] + mp_ref[1]
    h = h_ref[...]
    gi = jnp.dot(m, wih_ref[...], preferred_element_type=_f32) + bih_ref[...]
    gh = jnp.dot(h, whh_ref[...], preferred_element_type=_f32) + bhh_ref[...]
    i_r, i_z, i_n = gi[:, :H], gi[:, H:2 * H], gi[:, 2 * H:]
    h_r, h_z, h_n = gh[:, :H], gh[:, H:2 * H], gh[:, 2 * H:]
    r = jax.nn.sigmoid(i_r + h_r)
    z = jax.nn.sigmoid(i_z + h_z)
    n = jnp.tanh(i_n + r * h_n)
    h_new = (1.0 - z) * n + z * h
    o_ref[...] = h_new
    or_ref[...] = h_new.astype(_bf16).astype(_f32)


def _tc_gru(mparts, h, W_ih_t, W_hh_t, b_ih_row, b_hh_row):
    return pl.pallas_call(
        _gru_body,
        out_shape=(
            jax.ShapeDtypeStruct((N, H), _f32),
            jax.ShapeDtypeStruct((N, H), _f32),
        ),
    )(mparts, h, W_ih_t, W_hh_t, b_ih_row, b_hh_row)


def kernel(node_feat, edge_index, edge_feat, W_proj, b_proj, W_edge, b_edge,
           W_ih, W_hh, b_ih, b_hh):
    src = edge_index[0]
    dst = edge_index[1]

    S = jnp.kron(jnp.eye(H, dtype=_f32), jnp.ones((H, 1), dtype=_f32)).astype(_bf16)
    RT = jnp.kron(jnp.ones((1, H), dtype=_f32), jnp.eye(H, dtype=_f32)).astype(_bf16)

    zeros = jnp.zeros((N, H), dtype=_f32)

    A_b = _tc_edgeA(edge_feat, W_edge.T, b_edge.reshape(1, H * H))

    h, h_r = _tc_proj(node_feat, W_proj.T, b_proj.reshape(1, H))
    for _ in range(STEPS):
        h_src = _sc_gather(h_r, src)
        msg = _tc_msg(h_src, A_b, RT, S)
        mparts = _sc_scatter(msg, dst, zeros)
        h, h_r = _tc_gru(mparts, h, W_ih.T, W_hh.T, b_ih.reshape(1, 3 * H),
                         b_hh.reshape(1, 3 * H))
    return h


# speedup vs baseline: 1.5950x; 1.5950x over previous
import functools

import jax
import jax.numpy as jnp
from jax import lax
from jax.experimental import pallas as pl
from jax.experimental.pallas import tpu as pltpu
from jax.experimental.pallas import tpu_sc as plsc

N = 10000
E = 320000
NODE_DIM = 128
EDGE_DIM = 16
H = 16
STEPS = 3

NC = 2
NS = 16
NW = NC * NS
PER_TILE = E // NW
CHUNK = 1000

A_BLK = 8000
MSG_BLK = 8000

_f32 = jnp.float32
_bf16 = jnp.bfloat16

_SC_PARAMS = pltpu.CompilerParams(use_tc_tiling_on_sc=False)


def _gather_body(h_hbm, src_hbm, out_hbm, idx_v, rows_v, sem):
    wid = lax.axis_index("s") * NC + lax.axis_index("c")
    base = wid * PER_TILE

    @pl.loop(0, PER_TILE, step=CHUNK)
    def _(off):
        pltpu.sync_copy(src_hbm.at[pl.ds(base + off, CHUNK)], idx_v)
        pltpu.async_copy(h_hbm.at[idx_v], rows_v, sem).wait()
        pltpu.sync_copy(rows_v, out_hbm.at[pl.ds(base + off, CHUNK)])


def _sc_gather(h, src):
    mesh = plsc.VectorSubcoreMesh(core_axis_name="c", subcore_axis_name="s")
    k = pl.kernel(
        _gather_body,
        out_type=jax.ShapeDtypeStruct((E, H), _f32),
        mesh=mesh,
        scratch_types=[
            pltpu.VMEM((CHUNK,), jnp.int32),
            pltpu.VMEM((CHUNK, H), _f32),
            pltpu.SemaphoreType.DMA,
        ],
        compiler_params=_SC_PARAMS,
    )
    return k(h, src)


def _scatter_body(msg_hbm, dst_hbm, zero_hbm, out_hbm, idx_v, rows_v, acc_sh, sem):
    cid = lax.axis_index("c")
    sid = lax.axis_index("s")
    wid = sid * NC + cid

    @pl.when(sid == 0)
    def _():
        pltpu.sync_copy(zero_hbm, acc_sh)

    plsc.subcore_barrier()

    base = wid * PER_TILE

    @pl.loop(0, PER_TILE, step=CHUNK)
    def _(off):
        pltpu.sync_copy(dst_hbm.at[pl.ds(base + off, CHUNK)], idx_v)
        pltpu.sync_copy(msg_hbm.at[pl.ds(base + off, CHUNK)], rows_v)
        pltpu.sync_copy(rows_v, acc_sh.at[idx_v], add=True)

    plsc.subcore_barrier()

    rows_per = N // NS
    pltpu.sync_copy(
        acc_sh.at[pl.ds(sid * rows_per, rows_per)],
        out_hbm.at[cid, pl.ds(sid * rows_per, rows_per)],
    )


def _sc_scatter(msg, dst, zeros):
    mesh = plsc.VectorSubcoreMesh(core_axis_name="c", subcore_axis_name="s")
    k = pl.kernel(
        _scatter_body,
        out_type=jax.ShapeDtypeStruct((NC, N, H), _f32),
        mesh=mesh,
        scratch_types=[
            pltpu.VMEM((CHUNK,), jnp.int32),
            pltpu.VMEM((CHUNK, H), _f32),
            pltpu.VMEM_SHARED((N, H), _f32),
            pltpu.SemaphoreType.DMA,
        ],
        compiler_params=_SC_PARAMS,
    )
    return k(msg, dst, zeros)


def _proj_body(x_ref, w_ref, b_ref, o_ref, or_ref):
    h = jnp.dot(x_ref[...], w_ref[...], preferred_element_type=_f32) + b_ref[...]
    o_ref[...] = h
    or_ref[...] = h.astype(_bf16).astype(_f32)


def _tc_proj(node_feat, W_proj_t, b_proj_row):
    return pl.pallas_call(
        _proj_body,
        out_shape=(
            jax.ShapeDtypeStruct((N, H), _f32),
            jax.ShapeDtypeStruct((N, H), _f32),
        ),
    )(node_feat, W_proj_t, b_proj_row)


def _edgeA_body(ef_ref, w_ref, b_ref, o_ref):
    a = jnp.dot(ef_ref[...], w_ref[...], preferred_element_type=_f32) + b_ref[...]
    o_ref[...] = a.astype(_bf16)


def _tc_edgeA(ef2, W2, b2_row):
    grid = (E // A_BLK,)
    return pl.pallas_call(
        _edgeA_body,
        grid=grid,
        compiler_params=pltpu.CompilerParams(
            dimension_semantics=("parallel",)),
        in_specs=[
            pl.BlockSpec((A_BLK // 8, 8 * EDGE_DIM), lambda i: (i, 0)),
            pl.BlockSpec((8 * EDGE_DIM, 8 * H * H), lambda i: (0, 0)),
            pl.BlockSpec((1, 8 * H * H), lambda i: (0, 0)),
        ],
        out_specs=pl.BlockSpec((A_BLK // 8, 8 * H * H), lambda i: (i, 0)),
        out_shape=jax.ShapeDtypeStruct((E // 8, 8 * H * H), _bf16),
    )(ef2, W2, b2_row)


def _msg_body(hs_ref, a_ref, rt_ref, s_ref, o_ref):
    hs = hs_ref[...].astype(_bf16)
    a = a_ref[...]
    rt = rt_ref[...]
    s = s_ref[...]
    t = jnp.dot(hs, rt, preferred_element_type=_f32)
    p = a.astype(_f32) * t
    p_hi = p.astype(_bf16)
    p_lo = (p - p_hi.astype(_f32)).astype(_bf16)
    o_ref[...] = (
        jnp.dot(p_hi, s, preferred_element_type=_f32)
        + jnp.dot(p_lo, s, preferred_element_type=_f32)
    )


def _tc_msg(h_src2, A2, RT2, S2):
    grid = (E // MSG_BLK,)
    return pl.pallas_call(
        _msg_body,
        grid=grid,
        compiler_params=pltpu.CompilerParams(
            dimension_semantics=("parallel",)),
        in_specs=[
            pl.BlockSpec((MSG_BLK // 8, 8 * H), lambda i: (i, 0)),
            pl.BlockSpec((MSG_BLK // 8, 8 * H * H), lambda i: (i, 0)),
            pl.BlockSpec((8 * H, 8 * H * H), lambda i: (0, 0)),
            pl.BlockSpec((8 * H * H, 8 * H), lambda i: (0, 0)),
        ],
        out_specs=pl.BlockSpec((MSG_BLK // 8, 8 * H), lambda i: (i, 0)),
        out_shape=jax.ShapeDtypeStruct((E // 8, 8 * H), _f32),
    )(h_src2, A2, RT2, S2)


def _gru_body(mp_ref, h_ref, wih_ref, whh_ref, bih_ref, bhh_ref, o_ref, or_ref):
    m = mp_ref[0] + mp_ref[1]
    h = h_ref[...]
    gi = jnp.dot(m, wih_ref[...], preferred_element_type=_f32) + bih_ref[...]
    gh = jnp.dot(h, whh_ref[...], preferred_element_type=_f32) + bhh_ref[...]
    i_r, i_z, i_n = gi[:, :H], gi[:, H:2 * H], gi[:, 2 * H:]
    h_r, h_z, h_n = gh[:, :H], gh[:, H:2 * H], gh[:, 2 * H:]
    r = jax.nn.sigmoid(i_r + h_r)
    z = jax.nn.sigmoid(i_z + h_z)
    n = jnp.tanh(i_n + r * h_n)
    h_new = (1.0 - z) * n + z * h
    o_ref[...] = h_new
    or_ref[...] = h_new.astype(_bf16).astype(_f32)


def _tc_gru(mparts, h, W_ih_t, W_hh_t, b_ih_row, b_hh_row):
    return pl.pallas_call(
        _gru_body,
        out_shape=(
            jax.ShapeDtypeStruct((N, H), _f32),
            jax.ShapeDtypeStruct((N, H), _f32),
        ),
    )(mparts, h, W_ih_t, W_hh_t, b_ih_row, b_hh_row)


def kernel(node_feat, edge_index, edge_feat, W_proj, b_proj, W_edge, b_edge,
           W_ih, W_hh, b_ih, b_hh):
    src = edge_index[0]
    dst = edge_index[1]

    S = jnp.kron(jnp.eye(H, dtype=_f32), jnp.ones((H, 1), dtype=_f32))
    RT = jnp.kron(jnp.ones((1, H), dtype=_f32), jnp.eye(H, dtype=_f32))
    eye8 = jnp.eye(8, dtype=_f32)
    S2 = jnp.kron(eye8, S).astype(_bf16)
    RT2 = jnp.kron(eye8, RT).astype(_bf16)
    W2 = jnp.kron(eye8, W_edge.T)
    b2 = jnp.tile(b_edge, 8).reshape(1, 8 * H * H)
    ef2 = edge_feat.reshape(E // 8, 8 * EDGE_DIM)

    zeros = jnp.zeros((N, H), dtype=_f32)

    A2 = _tc_edgeA(ef2, W2, b2)

    h, h_r = _tc_proj(node_feat, W_proj.T, b_proj.reshape(1, H))
    for _ in range(STEPS):
        h_src2 = _sc_gather(h_r, src).reshape(E // 8, 8 * H)
        msg = _tc_msg(h_src2, A2, RT2, S2).reshape(E, H)
        mparts = _sc_scatter(msg, dst, zeros)
        h, h_r = _tc_gru(mparts, h, W_ih.T, W_hh.T, b_ih.reshape(1, 3 * H),
                         b_hh.reshape(1, 3 * H))
    return h


# speedup vs baseline: 1.6570x; 1.0388x over previous
import functools

import jax
import jax.numpy as jnp
from jax import lax
from jax.experimental import pallas as pl
from jax.experimental.pallas import tpu as pltpu
from jax.experimental.pallas import tpu_sc as plsc

N = 10000
E = 320000
NODE_DIM = 128
EDGE_DIM = 16
H = 16
STEPS = 3

NC = 2
NS = 16
NW = NC * NS
PER_TILE = E // NW
CHUNK = 1000

A_BLK = 8000
MSG_BLK = 8000

_f32 = jnp.float32
_bf16 = jnp.bfloat16

_SC_PARAMS = pltpu.CompilerParams(use_tc_tiling_on_sc=False)


def _gather_body(h_hbm, src_hbm, out_hbm, idx_v, rows_v, sem, *, per_tile):
    wid = lax.axis_index("s") * NC + lax.axis_index("c")
    base = wid * per_tile

    @pl.loop(0, per_tile, step=CHUNK)
    def _(off):
        pltpu.sync_copy(src_hbm.at[pl.ds(base + off, CHUNK)], idx_v)
        pltpu.async_copy(h_hbm.at[idx_v], rows_v, sem).wait()
        pltpu.sync_copy(rows_v, out_hbm.at[pl.ds(base + off, CHUNK)])


def _sc_gather(h, src, n_edges):
    mesh = plsc.VectorSubcoreMesh(core_axis_name="c", subcore_axis_name="s")
    k = pl.kernel(
        functools.partial(_gather_body, per_tile=n_edges // NW),
        out_type=jax.ShapeDtypeStruct((n_edges, H), _f32),
        mesh=mesh,
        scratch_types=[
            pltpu.VMEM((CHUNK,), jnp.int32),
            pltpu.VMEM((CHUNK, H), _f32),
            pltpu.SemaphoreType.DMA,
        ],
        compiler_params=_SC_PARAMS,
    )
    return k(h, src)


def _scatter_body(msg_hbm, dst_hbm, zero_hbm, out_hbm, idx_v, rows_v, acc_sh, sem, *, per_tile):
    cid = lax.axis_index("c")
    sid = lax.axis_index("s")
    wid = sid * NC + cid

    @pl.when(sid == 0)
    def _():
        pltpu.sync_copy(zero_hbm, acc_sh)

    plsc.subcore_barrier()

    base = wid * per_tile

    @pl.loop(0, per_tile, step=CHUNK)
    def _(off):
        pltpu.sync_copy(dst_hbm.at[pl.ds(base + off, CHUNK)], idx_v)
        pltpu.sync_copy(msg_hbm.at[pl.ds(base + off, CHUNK)], rows_v)
        pltpu.sync_copy(rows_v, acc_sh.at[idx_v], add=True)

    plsc.subcore_barrier()

    rows_per = N // NS
    pltpu.sync_copy(
        acc_sh.at[pl.ds(sid * rows_per, rows_per)],
        out_hbm.at[cid, pl.ds(sid * rows_per, rows_per)],
    )


def _sc_scatter(msg, dst, zeros, n_edges):
    mesh = plsc.VectorSubcoreMesh(core_axis_name="c", subcore_axis_name="s")
    k = pl.kernel(
        functools.partial(_scatter_body, per_tile=n_edges // NW),
        out_type=jax.ShapeDtypeStruct((NC, N, H), _f32),
        mesh=mesh,
        scratch_types=[
            pltpu.VMEM((CHUNK,), jnp.int32),
            pltpu.VMEM((CHUNK, H), _f32),
            pltpu.VMEM_SHARED((N, H), _f32),
            pltpu.SemaphoreType.DMA,
        ],
        compiler_params=_SC_PARAMS,
    )
    return k(msg, dst, zeros)


def _proj_body(x_ref, w_ref, b_ref, o_ref, or_ref):
    h = jnp.dot(x_ref[...], w_ref[...], preferred_element_type=_f32) + b_ref[...]
    o_ref[...] = h
    or_ref[...] = h.astype(_bf16).astype(_f32)


def _tc_proj(node_feat, W_proj_t, b_proj_row):
    return pl.pallas_call(
        _proj_body,
        out_shape=(
            jax.ShapeDtypeStruct((N, H), _f32),
            jax.ShapeDtypeStruct((N, H), _f32),
        ),
    )(node_feat, W_proj_t, b_proj_row)


def _edgeA_body(ef_ref, w_ref, b_ref, o_ref):
    a = jnp.dot(ef_ref[...], w_ref[...], preferred_element_type=_f32) + b_ref[...]
    o_ref[...] = a.astype(_bf16)


def _tc_edgeA(ef2, W2, b2_row):
    grid = (E // A_BLK,)
    return pl.pallas_call(
        _edgeA_body,
        grid=grid,
        compiler_params=pltpu.CompilerParams(
            dimension_semantics=("parallel",)),
        in_specs=[
            pl.BlockSpec((A_BLK // 8, 8 * EDGE_DIM), lambda i: (i, 0)),
            pl.BlockSpec((8 * EDGE_DIM, 8 * H * H), lambda i: (0, 0)),
            pl.BlockSpec((1, 8 * H * H), lambda i: (0, 0)),
        ],
        out_specs=pl.BlockSpec((A_BLK // 8, 8 * H * H), lambda i: (i, 0)),
        out_shape=jax.ShapeDtypeStruct((E // 8, 8 * H * H), _bf16),
    )(ef2, W2, b2_row)


def _msg_body(hs_ref, a_ref, rt_ref, s_ref, o_ref):
    hs = hs_ref[...].astype(_bf16)
    a = a_ref[...]
    rt = rt_ref[...]
    s = s_ref[...]
    t = jnp.dot(hs, rt, preferred_element_type=_f32)
    p = a.astype(_f32) * t
    p_hi = p.astype(_bf16)
    p_lo = (p - p_hi.astype(_f32)).astype(_bf16)
    o_ref[...] = (
        jnp.dot(p_hi, s, preferred_element_type=_f32)
        + jnp.dot(p_lo, s, preferred_element_type=_f32)
    )


def _tc_msg(h_src2, A2, RT2, S2, n_edges, a_off_blocks):
    grid = (n_edges // MSG_BLK,)
    return pl.pallas_call(
        _msg_body,
        grid=grid,
        compiler_params=pltpu.CompilerParams(
            dimension_semantics=("parallel",)),
        in_specs=[
            pl.BlockSpec((MSG_BLK // 8, 8 * H), lambda i: (i, 0)),
            pl.BlockSpec((MSG_BLK // 8, 8 * H * H),
                         lambda i: (i + a_off_blocks, 0)),
            pl.BlockSpec((8 * H, 8 * H * H), lambda i: (0, 0)),
            pl.BlockSpec((8 * H * H, 8 * H), lambda i: (0, 0)),
        ],
        out_specs=pl.BlockSpec((MSG_BLK // 8, 8 * H), lambda i: (i, 0)),
        out_shape=jax.ShapeDtypeStruct((n_edges // 8, 8 * H), _f32),
    )(h_src2, A2, RT2, S2)


def _gru_body(mp_ref, mq_ref, h_ref, wih_ref, whh_ref, bih_ref, bhh_ref, o_ref, or_ref):
    m = (mp_ref[0] + mp_ref[1]) + (mq_ref[0] + mq_ref[1])
    h = h_ref[...]
    gi = jnp.dot(m, wih_ref[...], preferred_element_type=_f32) + bih_ref[...]
    gh = jnp.dot(h, whh_ref[...], preferred_element_type=_f32) + bhh_ref[...]
    i_r, i_z, i_n = gi[:, :H], gi[:, H:2 * H], gi[:, 2 * H:]
    h_r, h_z, h_n = gh[:, :H], gh[:, H:2 * H], gh[:, 2 * H:]
    r = jax.nn.sigmoid(i_r + h_r)
    z = jax.nn.sigmoid(i_z + h_z)
    n = jnp.tanh(i_n + r * h_n)
    h_new = (1.0 - z) * n + z * h
    o_ref[...] = h_new
    or_ref[...] = h_new.astype(_bf16).astype(_f32)


def _tc_gru(mparts0, mparts1, h, W_ih_t, W_hh_t, b_ih_row, b_hh_row):
    return pl.pallas_call(
        _gru_body,
        out_shape=(
            jax.ShapeDtypeStruct((N, H), _f32),
            jax.ShapeDtypeStruct((N, H), _f32),
        ),
    )(mparts0, mparts1, h, W_ih_t, W_hh_t, b_ih_row, b_hh_row)


def kernel(node_feat, edge_index, edge_feat, W_proj, b_proj, W_edge, b_edge,
           W_ih, W_hh, b_ih, b_hh):
    src = edge_index[0]
    dst = edge_index[1]

    S = jnp.kron(jnp.eye(H, dtype=_f32), jnp.ones((H, 1), dtype=_f32))
    RT = jnp.kron(jnp.ones((1, H), dtype=_f32), jnp.eye(H, dtype=_f32))
    eye8 = jnp.eye(8, dtype=_f32)
    S2 = jnp.kron(eye8, S).astype(_bf16)
    RT2 = jnp.kron(eye8, RT).astype(_bf16)
    W2 = jnp.kron(eye8, W_edge.T)
    b2 = jnp.tile(b_edge, 8).reshape(1, 8 * H * H)
    ef2 = edge_feat.reshape(E // 8, 8 * EDGE_DIM)

    zeros = jnp.zeros((N, H), dtype=_f32)

    A2 = _tc_edgeA(ef2, W2, b2)

    E2 = E // 2
    src0, src1 = src[:E2], src[E2:]
    dst0, dst1 = dst[:E2], dst[E2:]

    h, h_r = _tc_proj(node_feat, W_proj.T, b_proj.reshape(1, H))
    for _ in range(STEPS):
        g0 = _sc_gather(h_r, src0, E2).reshape(E2 // 8, 8 * H)
        g1 = _sc_gather(h_r, src1, E2).reshape(E2 // 8, 8 * H)
        msg0 = _tc_msg(g0, A2, RT2, S2, E2, 0).reshape(E2, H)
        mp0 = _sc_scatter(msg0, dst0, zeros, E2)
        msg1 = _tc_msg(g1, A2, RT2, S2, E2, E2 // MSG_BLK).reshape(E2, H)
        mp1 = _sc_scatter(msg1, dst1, zeros, E2)
        h, h_r = _tc_gru(mp0, mp1, h, W_ih.T, W_hh.T, b_ih.reshape(1, 3 * H),
                         b_hh.reshape(1, 3 * H))
    return h
